# baseline (speedup 1.0000x reference)
import functools

import jax
import jax.numpy as jnp
from jax import lax
from jax.experimental import pallas as pl
from jax.experimental.pallas import tpu as pltpu
from jax.experimental.pallas import tpu_sc as plsc

N = 10000
NPAD = 10240
H = 128
E = 320000
SLOPE = 0.1

NC = 2
NS = 16
NW = NC * NS
C = 128
CW = 80
E_PAD = NW * CW * C
STRIPE = NPAD // NS
BR = 2048


def _leaky(v):
    return jnp.where(v >= 0, v, SLOPE * v)


def _embed(x, W1, b1, W2, b2):
    n = x.shape[0]

    def body(x_ref, w1_ref, b1_ref, w2_ref, b2_ref, o_ref):
        h = _leaky(jnp.dot(x_ref[...], w1_ref[...],
                           preferred_element_type=jnp.float32) + b1_ref[...])
        o_ref[...] = _leaky(jnp.dot(h, w2_ref[...],
                                    preferred_element_type=jnp.float32) + b2_ref[...])

    return pl.pallas_call(
        body,
        out_shape=jax.ShapeDtypeStruct((n, H), jnp.float32),
    )(x, W1, b1.reshape(1, H), W2, b2.reshape(1, H))


def _layer(p, x, Wl, Wr, b, last):
    dout = Wl.shape[1]

    def body(p_ref, x_ref, wl_ref, wr_ref, b_ref, o_ref):
        pv = p_ref[...]
        agg = pv[0] + pv[1]
        s = (jnp.dot(agg, wl_ref[...], preferred_element_type=jnp.float32)
             + jnp.dot(x_ref[...], wr_ref[...], preferred_element_type=jnp.float32)
             + b_ref[...])
        if last:
            o_ref[...] = 1.0 / (1.0 + jnp.exp(-s))
        else:
            o_ref[...] = _leaky(s)

    grid = (NPAD // BR,)
    return pl.pallas_call(
        body,
        grid=grid,
        in_specs=[
            pl.BlockSpec((2, BR, H), lambda i: (0, i, 0)),
            pl.BlockSpec((BR, H), lambda i: (i, 0)),
            pl.BlockSpec((H, dout), lambda i: (0, 0)),
            pl.BlockSpec((H, dout), lambda i: (0, 0)),
            pl.BlockSpec((1, dout), lambda i: (0, 0)),
        ],
        out_specs=pl.BlockSpec((BR, dout), lambda i: (i, 0)),
        out_shape=jax.ShapeDtypeStruct((NPAD, dout), jnp.float32),
    )(p, x, Wl, Wr, b.reshape(1, dout))


def _make_segsum(D):
    ZR = 64

    mesh = plsc.VectorSubcoreMesh(
        core_axis_name="c", subcore_axis_name="s",
        num_cores=NC, num_subcores=NS)

    @functools.partial(
        pl.kernel,
        mesh=mesh,
        out_type=jax.ShapeDtypeStruct((NC * NPAD, D), jnp.float32),
        scratch_types=[
            pltpu.VMEM((2, C), jnp.int32),
            pltpu.VMEM((2, C), jnp.int32),
            pltpu.VMEM((C, D), jnp.float32),
            pltpu.VMEM((C, D), jnp.float32),
            pltpu.VMEM((ZR, D), jnp.float32),
            pltpu.VMEM_SHARED((NPAD, D), jnp.float32),
            pltpu.SemaphoreType.DMA,
            pltpu.SemaphoreType.DMA,
            pltpu.SemaphoreType.DMA,
            pltpu.SemaphoreType.DMA,
        ],
    )
    def seg(y_hbm, eidx_hbm, out_hbm,
            ib0, ib1, rows0, rows1, zbuf, acc, si0, si1, sg0, sg1):
        cid = lax.axis_index("c")
        sid = lax.axis_index("s")
        w = sid * NC + cid
        cbase = w * CW

        for i in range(ZR):
            for j in range(D // 16):
                zbuf[i, pl.ds(j * 16, 16)] = jnp.zeros((16,), jnp.float32)
        base = sid * STRIPE
        zcps = [pltpu.async_copy(zbuf, acc.at[pl.ds(base + r * ZR, ZR)], sg0)
                for r in range(STRIPE // ZR)]
        for cp in zcps:
            cp.wait()
        plsc.subcore_barrier()

        pltpu.async_copy(eidx_hbm.at[cbase], ib0, si0)
        pltpu.async_copy(eidx_hbm.at[cbase + 1], ib1, si1)
        pltpu.make_async_copy(eidx_hbm.at[cbase], ib0, si0).wait()
        pltpu.async_copy(y_hbm.at[ib0.at[0]], rows0, sg0)
        pltpu.make_async_copy(eidx_hbm.at[cbase + 1], ib1, si1).wait()
        pltpu.async_copy(y_hbm.at[ib1.at[0]], rows1, sg1)

        def body(j, carry):
            a = cbase + 2 * j
            pltpu.make_async_copy(y_hbm.at[ib0.at[0]], rows0, sg0).wait()
            pltpu.sync_copy(rows0, acc.at[ib0.at[1]], add=True)
            pltpu.async_copy(eidx_hbm.at[a + 2], ib0, si0)
            pltpu.make_async_copy(y_hbm.at[ib1.at[0]], rows1, sg1).wait()
            pltpu.sync_copy(rows1, acc.at[ib1.at[1]], add=True)
            pltpu.async_copy(eidx_hbm.at[a + 3], ib1, si1)
            pltpu.make_async_copy(eidx_hbm.at[a + 2], ib0, si0).wait()
            pltpu.async_copy(y_hbm.at[ib0.at[0]], rows0, sg0)
            pltpu.make_async_copy(eidx_hbm.at[a + 3], ib1, si1).wait()
            pltpu.async_copy(y_hbm.at[ib1.at[0]], rows1, sg1)
            return carry

        lax.fori_loop(0, CW // 2 - 1, body, 0)
        pltpu.make_async_copy(y_hbm.at[ib0.at[0]], rows0, sg0).wait()
        pltpu.sync_copy(rows0, acc.at[ib0.at[1]], add=True)
        pltpu.make_async_copy(y_hbm.at[ib1.at[0]], rows1, sg1).wait()
        pltpu.sync_copy(rows1, acc.at[ib1.at[1]], add=True)

        plsc.subcore_barrier()
        pltpu.sync_copy(acc.at[pl.ds(base, STRIPE)],
                        out_hbm.at[pl.ds(cid * NPAD + base, STRIPE)])

    return seg


_seg_cache = {}


def _seg128(y, eidx):
    if H not in _seg_cache:
        _seg_cache[H] = _make_segsum(H)
    return _seg_cache[H](y, eidx).reshape(2, NPAD, H)


def kernel(x_gen, x_load, x_or, x_ex, edge_index, object_ptv,
           Wg1, bg1, Wg2, bg2, Wl1, bl1, Wl2, bl2,
           Wo1, bo1, Wo2, bo2, We1, be1, We2, be2,
           sage0_Wl, sage0_Wr, sage0_b,
           sage1_Wl, sage1_Wr, sage1_b,
           sage2_Wl, sage2_Wr, sage2_b,
           sage3_Wl, sage3_Wr, sage3_b):
    f32 = jnp.float32

    def pad_k(x, w):
        k = x.shape[1]
        return (jnp.pad(x, ((0, 0), (0, 8 - k))), jnp.pad(w, ((0, 8 - k), (0, 0))))

    xg, wg1 = pad_k(x_gen, Wg1)
    xl, wl1 = pad_k(x_load, Wl1)
    xo, wo1 = pad_k(jnp.pad(x_or, ((0, 4), (0, 0))), Wo1)
    xe, we1 = pad_k(jnp.pad(x_ex, ((0, 4), (0, 0))), We1)
    eg = _embed(xg, wg1, bg1, Wg2, bg2)
    el = _embed(xl, wl1, bl1, Wl2, bl2)
    eo = _embed(xo, wo1, bo1, Wo2, bo2)[:3500]
    ee = _embed(xe, we1, be1, We2, be2)[:3500]
    x0 = jnp.concatenate(
        [eg, el, eo, ee, jnp.zeros((NPAD - N, H), f32)], axis=0)

    src = jnp.concatenate([edge_index[0], jnp.full((E_PAD - E,), N, jnp.int32)])
    dst = jnp.concatenate([edge_index[1], jnp.full((E_PAD - E,), N, jnp.int32)])
    eidx = jnp.stack([src.reshape(NW * CW, C), dst.reshape(NW * CW, C)], axis=1)

    x1 = _layer(_seg128(x0, eidx), x0, sage0_Wl, sage0_Wr, sage0_b, False)
    x2 = _layer(_seg128(x1, eidx), x1, sage1_Wl, sage1_Wr, sage1_b, False)
    x3 = _layer(_seg128(x2, eidx), x2, sage2_Wl, sage2_Wr, sage2_b, False)
    wl3 = jnp.pad(sage3_Wl, ((0, 0), (0, 7)))
    wr3 = jnp.pad(sage3_Wr, ((0, 0), (0, 7)))
    b3 = jnp.pad(sage3_b, ((0, 7),))
    out = _layer(_seg128(x3, eidx), x3, wl3, wr3, b3, True)
    return out[:N, :1]


# speedup vs baseline: 1.4549x; 1.4549x over previous
import functools

import jax
import jax.numpy as jnp
from jax import lax
from jax.experimental import pallas as pl
from jax.experimental.pallas import tpu as pltpu
from jax.experimental.pallas import tpu_sc as plsc

N = 10000
NPAD = 10240
H = 128
E = 320000
SLOPE = 0.1

NC = 2
NS = 16
NW = NC * NS
C = 128
CW = 80
E_PAD = NW * CW * C
STRIPE = NPAD // NS
BR = 2048


def _leaky(v):
    return jnp.where(v >= 0, v, SLOPE * v)


def _embed(x, W1, b1, W2, b2):
    n = x.shape[0]

    def body(x_ref, w1_ref, b1_ref, w2_ref, b2_ref, o_ref):
        h = _leaky(jnp.dot(x_ref[...], w1_ref[...],
                           preferred_element_type=jnp.float32) + b1_ref[...])
        o_ref[...] = _leaky(jnp.dot(h, w2_ref[...],
                                    preferred_element_type=jnp.float32) + b2_ref[...])

    return pl.pallas_call(
        body,
        out_shape=jax.ShapeDtypeStruct((n, H), jnp.float32),
    )(x, W1, b1.reshape(1, H), W2, b2.reshape(1, H))


def _layer(p, x, Wl, Wr, b, last):
    dout = Wl.shape[1]

    def body(p_ref, x_ref, wl_ref, wr_ref, b_ref, o_ref):
        pv = p_ref[...]
        agg = pv[0] + pv[1]
        s = (jnp.dot(agg, wl_ref[...], preferred_element_type=jnp.float32)
             + jnp.dot(x_ref[...], wr_ref[...], preferred_element_type=jnp.float32)
             + b_ref[...])
        if last:
            o_ref[...] = 1.0 / (1.0 + jnp.exp(-s))
        else:
            o_ref[...] = _leaky(s)

    grid = (NPAD // BR,)
    return pl.pallas_call(
        body,
        grid=grid,
        in_specs=[
            pl.BlockSpec((2, BR, H), lambda i: (0, i, 0)),
            pl.BlockSpec((BR, H), lambda i: (i, 0)),
            pl.BlockSpec((H, dout), lambda i: (0, 0)),
            pl.BlockSpec((H, dout), lambda i: (0, 0)),
            pl.BlockSpec((1, dout), lambda i: (0, 0)),
        ],
        out_specs=pl.BlockSpec((BR, dout), lambda i: (i, 0)),
        out_shape=jax.ShapeDtypeStruct((NPAD, dout), jnp.float32),
    )(p, x, Wl, Wr, b.reshape(1, dout))


def _make_segsum(D):
    ZR = 64

    mesh = plsc.VectorSubcoreMesh(
        core_axis_name="c", subcore_axis_name="s",
        num_cores=NC, num_subcores=NS)

    @functools.partial(
        pl.kernel,
        mesh=mesh,
        out_type=jax.ShapeDtypeStruct((NC * NPAD, D), jnp.float32),
        scratch_types=[
            pltpu.VMEM((2, C), jnp.int32),
            pltpu.VMEM((2, C), jnp.int32),
            pltpu.VMEM((C, D), jnp.float32),
            pltpu.VMEM((C, D), jnp.float32),
            pltpu.VMEM((ZR, D), jnp.float32),
            pltpu.VMEM_SHARED((NPAD, D), jnp.float32),
            pltpu.SemaphoreType.DMA,
            pltpu.SemaphoreType.DMA,
            pltpu.SemaphoreType.DMA,
            pltpu.SemaphoreType.DMA,
            pltpu.SemaphoreType.DMA,
            pltpu.SemaphoreType.DMA,
        ],
    )
    def seg(y_hbm, eidx_hbm, out_hbm,
            ib0, ib1, rows0, rows1, zbuf, acc, si0, si1, sg0, sg1, ss0, ss1):
        cid = lax.axis_index("c")
        sid = lax.axis_index("s")
        w = sid * NC + cid
        cbase = w * CW

        for i in range(ZR):
            for j in range(D // 16):
                zbuf[i, pl.ds(j * 16, 16)] = jnp.zeros((16,), jnp.float32)
        base = sid * STRIPE
        zcps = [pltpu.async_copy(zbuf, acc.at[pl.ds(base + r * ZR, ZR)], sg0)
                for r in range(STRIPE // ZR)]
        for cp in zcps:
            cp.wait()
        plsc.subcore_barrier()

        pltpu.async_copy(eidx_hbm.at[cbase], ib0, si0)
        pltpu.async_copy(eidx_hbm.at[cbase + 1], ib1, si1)
        pltpu.make_async_copy(eidx_hbm.at[cbase], ib0, si0).wait()
        pltpu.async_copy(y_hbm.at[ib0.at[0]], rows0, sg0)
        pltpu.make_async_copy(eidx_hbm.at[cbase + 1], ib1, si1).wait()
        pltpu.async_copy(y_hbm.at[ib1.at[0]], rows1, sg1)

        def body(j, carry):
            a = cbase + 2 * j
            pltpu.make_async_copy(y_hbm.at[ib0.at[0]], rows0, sg0).wait()
            pltpu.async_copy(rows0, acc.at[ib0.at[1]], ss0, add=True)
            pltpu.make_async_copy(y_hbm.at[ib1.at[0]], rows1, sg1).wait()
            pltpu.async_copy(rows1, acc.at[ib1.at[1]], ss1, add=True)
            pltpu.make_async_copy(rows0, acc.at[ib0.at[1]], ss0).wait()
            pltpu.async_copy(eidx_hbm.at[a + 2], ib0, si0)
            pltpu.make_async_copy(eidx_hbm.at[a + 2], ib0, si0).wait()
            pltpu.async_copy(y_hbm.at[ib0.at[0]], rows0, sg0)
            pltpu.make_async_copy(rows1, acc.at[ib1.at[1]], ss1).wait()
            pltpu.async_copy(eidx_hbm.at[a + 3], ib1, si1)
            pltpu.make_async_copy(eidx_hbm.at[a + 3], ib1, si1).wait()
            pltpu.async_copy(y_hbm.at[ib1.at[0]], rows1, sg1)
            return carry

        lax.fori_loop(0, CW // 2 - 1, body, 0)
        pltpu.make_async_copy(y_hbm.at[ib0.at[0]], rows0, sg0).wait()
        pltpu.async_copy(rows0, acc.at[ib0.at[1]], ss0, add=True)
        pltpu.make_async_copy(y_hbm.at[ib1.at[0]], rows1, sg1).wait()
        pltpu.async_copy(rows1, acc.at[ib1.at[1]], ss1, add=True)
        pltpu.make_async_copy(rows0, acc.at[ib0.at[1]], ss0).wait()
        pltpu.make_async_copy(rows1, acc.at[ib1.at[1]], ss1).wait()

        plsc.subcore_barrier()
        pltpu.sync_copy(acc.at[pl.ds(base, STRIPE)],
                        out_hbm.at[pl.ds(cid * NPAD + base, STRIPE)])

    return seg


_seg_cache = {}


def _seg128(y, eidx):
    if H not in _seg_cache:
        _seg_cache[H] = _make_segsum(H)
    return _seg_cache[H](y, eidx).reshape(2, NPAD, H)


def kernel(x_gen, x_load, x_or, x_ex, edge_index, object_ptv,
           Wg1, bg1, Wg2, bg2, Wl1, bl1, Wl2, bl2,
           Wo1, bo1, Wo2, bo2, We1, be1, We2, be2,
           sage0_Wl, sage0_Wr, sage0_b,
           sage1_Wl, sage1_Wr, sage1_b,
           sage2_Wl, sage2_Wr, sage2_b,
           sage3_Wl, sage3_Wr, sage3_b):
    f32 = jnp.float32

    def pad_k(x, w):
        k = x.shape[1]
        return (jnp.pad(x, ((0, 0), (0, 8 - k))), jnp.pad(w, ((0, 8 - k), (0, 0))))

    xg, wg1 = pad_k(x_gen, Wg1)
    xl, wl1 = pad_k(x_load, Wl1)
    xo, wo1 = pad_k(jnp.pad(x_or, ((0, 4), (0, 0))), Wo1)
    xe, we1 = pad_k(jnp.pad(x_ex, ((0, 4), (0, 0))), We1)
    eg = _embed(xg, wg1, bg1, Wg2, bg2)
    el = _embed(xl, wl1, bl1, Wl2, bl2)
    eo = _embed(xo, wo1, bo1, Wo2, bo2)[:3500]
    ee = _embed(xe, we1, be1, We2, be2)[:3500]
    x0 = jnp.concatenate(
        [eg, el, eo, ee, jnp.zeros((NPAD - N, H), f32)], axis=0)

    src = jnp.concatenate([edge_index[0], jnp.full((E_PAD - E,), N, jnp.int32)])
    dst = jnp.concatenate([edge_index[1], jnp.full((E_PAD - E,), N, jnp.int32)])
    eidx = jnp.stack([src.reshape(NW * CW, C), dst.reshape(NW * CW, C)], axis=1)

    x1 = _layer(_seg128(x0, eidx), x0, sage0_Wl, sage0_Wr, sage0_b, False)
    x2 = _layer(_seg128(x1, eidx), x1, sage1_Wl, sage1_Wr, sage1_b, False)
    x3 = _layer(_seg128(x2, eidx), x2, sage2_Wl, sage2_Wr, sage2_b, False)
    wl3 = jnp.pad(sage3_Wl, ((0, 0), (0, 7)))
    wr3 = jnp.pad(sage3_Wr, ((0, 0), (0, 7)))
    b3 = jnp.pad(sage3_b, ((0, 7),))
    out = _layer(_seg128(x3, eidx), x3, wl3, wr3, b3, True)
    return out[:N, :1]


# speedup vs baseline: 1.4568x; 1.0013x over previous
import functools

import jax
import jax.numpy as jnp
from jax import lax
from jax.experimental import pallas as pl
from jax.experimental.pallas import tpu as pltpu
from jax.experimental.pallas import tpu_sc as plsc

N = 10000
NPAD = 10240
H = 128
E = 320000
SLOPE = 0.1

NC = 2
NS = 16
NW = NC * NS
C = 128
CW = 80
E_PAD = NW * CW * C
STRIPE = NPAD // NS
BR = 2048


def _leaky(v):
    return jnp.where(v >= 0, v, SLOPE * v)


def _embed(x, W1, b1, W2, b2):
    n = x.shape[0]

    def body(x_ref, w1_ref, b1_ref, w2_ref, b2_ref, o_ref):
        h = _leaky(jnp.dot(x_ref[...], w1_ref[...],
                           preferred_element_type=jnp.float32) + b1_ref[...])
        o_ref[...] = _leaky(jnp.dot(h, w2_ref[...],
                                    preferred_element_type=jnp.float32) + b2_ref[...])

    return pl.pallas_call(
        body,
        out_shape=jax.ShapeDtypeStruct((n, H), jnp.float32),
    )(x, W1, b1.reshape(1, H), W2, b2.reshape(1, H))


def _layer(p, x, Wl, Wr, b, last):
    dout = Wl.shape[1]

    def body(p_ref, x_ref, wl_ref, wr_ref, b_ref, o_ref):
        pv = p_ref[...]
        agg = pv[0] + pv[1]
        s = (jnp.dot(agg, wl_ref[...], preferred_element_type=jnp.float32)
             + jnp.dot(x_ref[...], wr_ref[...], preferred_element_type=jnp.float32)
             + b_ref[...])
        if last:
            o_ref[...] = 1.0 / (1.0 + jnp.exp(-s))
        else:
            o_ref[...] = _leaky(s)

    grid = (NPAD // BR,)
    return pl.pallas_call(
        body,
        grid=grid,
        in_specs=[
            pl.BlockSpec((2, BR, H), lambda i: (0, i, 0)),
            pl.BlockSpec((BR, H), lambda i: (i, 0)),
            pl.BlockSpec((H, dout), lambda i: (0, 0)),
            pl.BlockSpec((H, dout), lambda i: (0, 0)),
            pl.BlockSpec((1, dout), lambda i: (0, 0)),
        ],
        out_specs=pl.BlockSpec((BR, dout), lambda i: (i, 0)),
        out_shape=jax.ShapeDtypeStruct((NPAD, dout), jnp.float32),
    )(p, x, Wl, Wr, b.reshape(1, dout))


def _make_segsum(D):
    ZR = 64

    mesh = plsc.VectorSubcoreMesh(
        core_axis_name="c", subcore_axis_name="s",
        num_cores=NC, num_subcores=NS)

    @functools.partial(
        pl.kernel,
        mesh=mesh,
        out_type=jax.ShapeDtypeStruct((NC * NPAD, D), jnp.float32),
        scratch_types=[
            pltpu.VMEM((2, C), jnp.int32),
            pltpu.VMEM((2, C), jnp.int32),
            pltpu.VMEM((C, D), jnp.float32),
            pltpu.VMEM((C, D), jnp.float32),
            pltpu.VMEM((ZR, D), jnp.float32),
            pltpu.VMEM_SHARED((NPAD, D), jnp.float32),
            pltpu.SemaphoreType.DMA,
            pltpu.SemaphoreType.DMA,
            pltpu.SemaphoreType.DMA,
            pltpu.SemaphoreType.DMA,
            pltpu.SemaphoreType.DMA,
            pltpu.SemaphoreType.DMA,
        ],
    )
    def seg(y_hbm, eidx_hbm, out_hbm,
            ib0, ib1, rows0, rows1, zbuf, acc, si0, si1, sg0, sg1, ss0, ss1):
        cid = lax.axis_index("c")
        sid = lax.axis_index("s")
        w = sid * NC + cid
        cbase = w * CW

        for i in range(ZR):
            for j in range(D // 16):
                zbuf[i, pl.ds(j * 16, 16)] = jnp.zeros((16,), jnp.float32)
        base = sid * STRIPE
        zcps = [pltpu.async_copy(zbuf, acc.at[pl.ds(base + r * ZR, ZR)], sg0)
                for r in range(STRIPE // ZR)]
        for cp in zcps:
            cp.wait()
        plsc.subcore_barrier()

        pltpu.async_copy(eidx_hbm.at[cbase], ib0, si0)
        pltpu.async_copy(eidx_hbm.at[cbase + 1], ib1, si1)
        pltpu.make_async_copy(eidx_hbm.at[cbase], ib0, si0).wait()
        pltpu.async_copy(y_hbm.at[ib0.at[0]], rows0, sg0)
        pltpu.make_async_copy(eidx_hbm.at[cbase + 1], ib1, si1).wait()
        pltpu.async_copy(y_hbm.at[ib1.at[0]], rows1, sg1)

        def body(j, carry):
            a = cbase + 2 * j
            pltpu.make_async_copy(y_hbm.at[ib0.at[0]], rows0, sg0).wait()
            pltpu.async_copy(rows0, acc.at[ib0.at[1]], ss0, add=True)
            pltpu.make_async_copy(y_hbm.at[ib1.at[0]], rows1, sg1).wait()
            pltpu.async_copy(rows1, acc.at[ib1.at[1]], ss1, add=True)
            pltpu.make_async_copy(rows0, acc.at[ib0.at[1]], ss0).wait()
            pltpu.async_copy(eidx_hbm.at[a + 2], ib0, si0)
            pltpu.make_async_copy(eidx_hbm.at[a + 2], ib0, si0).wait()
            pltpu.async_copy(y_hbm.at[ib0.at[0]], rows0, sg0)
            pltpu.make_async_copy(rows1, acc.at[ib1.at[1]], ss1).wait()
            pltpu.async_copy(eidx_hbm.at[a + 3], ib1, si1)
            pltpu.make_async_copy(eidx_hbm.at[a + 3], ib1, si1).wait()
            pltpu.async_copy(y_hbm.at[ib1.at[0]], rows1, sg1)
            return carry

        lax.fori_loop(0, CW // 2 - 1, body, 0)
        pltpu.make_async_copy(y_hbm.at[ib0.at[0]], rows0, sg0).wait()
        pltpu.async_copy(rows0, acc.at[ib0.at[1]], ss0, add=True)
        pltpu.make_async_copy(y_hbm.at[ib1.at[0]], rows1, sg1).wait()
        pltpu.async_copy(rows1, acc.at[ib1.at[1]], ss1, add=True)
        pltpu.make_async_copy(rows0, acc.at[ib0.at[1]], ss0).wait()
        pltpu.make_async_copy(rows1, acc.at[ib1.at[1]], ss1).wait()

        plsc.subcore_barrier()
        pltpu.sync_copy(acc.at[pl.ds(base, STRIPE)],
                        out_hbm.at[pl.ds(cid * NPAD + base, STRIPE)])

    return seg


def _make_segsum1d():
    mesh = plsc.VectorSubcoreMesh(
        core_axis_name="c", subcore_axis_name="s",
        num_cores=NC, num_subcores=NS)

    @functools.partial(
        pl.kernel,
        mesh=mesh,
        out_type=jax.ShapeDtypeStruct((NC * NPAD,), jnp.float32),
        scratch_types=[
            pltpu.VMEM((2, C), jnp.int32),
            pltpu.VMEM((2, C), jnp.int32),
            pltpu.VMEM((C,), jnp.float32),
            pltpu.VMEM((C,), jnp.float32),
            pltpu.VMEM((STRIPE,), jnp.float32),
            pltpu.VMEM_SHARED((NPAD,), jnp.float32),
            pltpu.SemaphoreType.DMA,
            pltpu.SemaphoreType.DMA,
            pltpu.SemaphoreType.DMA,
            pltpu.SemaphoreType.DMA,
            pltpu.SemaphoreType.DMA,
            pltpu.SemaphoreType.DMA,
        ],
    )
    def seg(y_hbm, eidx_hbm, out_hbm,
            ib0, ib1, rows0, rows1, zbuf, acc, si0, si1, sg0, sg1, ss0, ss1):
        cid = lax.axis_index("c")
        sid = lax.axis_index("s")
        w = sid * NC + cid
        cbase = w * CW

        for j in range(STRIPE // 16):
            zbuf[pl.ds(j * 16, 16)] = jnp.zeros((16,), jnp.float32)
        base = sid * STRIPE
        pltpu.sync_copy(zbuf, acc.at[pl.ds(base, STRIPE)])
        plsc.subcore_barrier()

        pltpu.async_copy(eidx_hbm.at[cbase], ib0, si0)
        pltpu.async_copy(eidx_hbm.at[cbase + 1], ib1, si1)
        pltpu.make_async_copy(eidx_hbm.at[cbase], ib0, si0).wait()
        pltpu.async_copy(y_hbm.at[ib0.at[0]], rows0, sg0)
        pltpu.make_async_copy(eidx_hbm.at[cbase + 1], ib1, si1).wait()
        pltpu.async_copy(y_hbm.at[ib1.at[0]], rows1, sg1)

        def body(j, carry):
            a = cbase + 2 * j
            pltpu.make_async_copy(y_hbm.at[ib0.at[0]], rows0, sg0).wait()
            pltpu.async_copy(rows0, acc.at[ib0.at[1]], ss0, add=True)
            pltpu.make_async_copy(y_hbm.at[ib1.at[0]], rows1, sg1).wait()
            pltpu.async_copy(rows1, acc.at[ib1.at[1]], ss1, add=True)
            pltpu.make_async_copy(rows0, acc.at[ib0.at[1]], ss0).wait()
            pltpu.async_copy(eidx_hbm.at[a + 2], ib0, si0)
            pltpu.make_async_copy(eidx_hbm.at[a + 2], ib0, si0).wait()
            pltpu.async_copy(y_hbm.at[ib0.at[0]], rows0, sg0)
            pltpu.make_async_copy(rows1, acc.at[ib1.at[1]], ss1).wait()
            pltpu.async_copy(eidx_hbm.at[a + 3], ib1, si1)
            pltpu.make_async_copy(eidx_hbm.at[a + 3], ib1, si1).wait()
            pltpu.async_copy(y_hbm.at[ib1.at[0]], rows1, sg1)
            return carry

        lax.fori_loop(0, CW // 2 - 1, body, 0)
        pltpu.make_async_copy(y_hbm.at[ib0.at[0]], rows0, sg0).wait()
        pltpu.async_copy(rows0, acc.at[ib0.at[1]], ss0, add=True)
        pltpu.make_async_copy(y_hbm.at[ib1.at[0]], rows1, sg1).wait()
        pltpu.async_copy(rows1, acc.at[ib1.at[1]], ss1, add=True)
        pltpu.make_async_copy(rows0, acc.at[ib0.at[1]], ss0).wait()
        pltpu.make_async_copy(rows1, acc.at[ib1.at[1]], ss1).wait()

        plsc.subcore_barrier()
        pltpu.sync_copy(acc.at[pl.ds(base, STRIPE)],
                        out_hbm.at[pl.ds(cid * NPAD + base, STRIPE)])

    return seg


_seg_cache = {}


def _seg128(y, eidx):
    if H not in _seg_cache:
        _seg_cache[H] = _make_segsum(H)
    return _seg_cache[H](y, eidx).reshape(2, NPAD, H)


def _seg1d(y, eidx):
    if 1 not in _seg_cache:
        _seg_cache[1] = _make_segsum1d()
    return _seg_cache[1](y, eidx)


def _head3(x, Wl, Wr, b):
    dout = Wl.shape[1]

    def body(x_ref, wl_ref, wr_ref, b_ref, y_ref, z_ref):
        xv = x_ref[...]
        y_ref[...] = jnp.dot(xv, wl_ref[...], preferred_element_type=jnp.float32,
                             precision=jax.lax.Precision.HIGHEST)
        z_ref[...] = jnp.dot(xv, wr_ref[...], preferred_element_type=jnp.float32,
                             precision=jax.lax.Precision.HIGHEST) + b_ref[...]

    grid = (NPAD // BR,)
    return pl.pallas_call(
        body,
        grid=grid,
        in_specs=[
            pl.BlockSpec((BR, H), lambda i: (i, 0)),
            pl.BlockSpec((H, dout), lambda i: (0, 0)),
            pl.BlockSpec((H, dout), lambda i: (0, 0)),
            pl.BlockSpec((1, dout), lambda i: (0, 0)),
        ],
        out_specs=[
            pl.BlockSpec((BR, dout), lambda i: (i, 0)),
            pl.BlockSpec((BR, dout), lambda i: (i, 0)),
        ],
        out_shape=[jax.ShapeDtypeStruct((NPAD, dout), jnp.float32)] * 2,
    )(x, Wl, Wr, b.reshape(1, dout))


def _final1d(p, z):
    def body(p_ref, z_ref, o_ref):
        pv = p_ref[...]
        s = pv[0] + pv[1] + z_ref[...]
        o_ref[...] = 1.0 / (1.0 + jnp.exp(-s))

    return pl.pallas_call(
        body,
        out_shape=jax.ShapeDtypeStruct(z.shape, jnp.float32),
    )(p, z)


def kernel(x_gen, x_load, x_or, x_ex, edge_index, object_ptv,
           Wg1, bg1, Wg2, bg2, Wl1, bl1, Wl2, bl2,
           Wo1, bo1, Wo2, bo2, We1, be1, We2, be2,
           sage0_Wl, sage0_Wr, sage0_b,
           sage1_Wl, sage1_Wr, sage1_b,
           sage2_Wl, sage2_Wr, sage2_b,
           sage3_Wl, sage3_Wr, sage3_b):
    f32 = jnp.float32

    def pad_k(x, w):
        k = x.shape[1]
        return (jnp.pad(x, ((0, 0), (0, 8 - k))), jnp.pad(w, ((0, 8 - k), (0, 0))))

    xg, wg1 = pad_k(x_gen, Wg1)
    xl, wl1 = pad_k(x_load, Wl1)
    xo, wo1 = pad_k(jnp.pad(x_or, ((0, 4), (0, 0))), Wo1)
    xe, we1 = pad_k(jnp.pad(x_ex, ((0, 4), (0, 0))), We1)
    eg = _embed(xg, wg1, bg1, Wg2, bg2)
    el = _embed(xl, wl1, bl1, Wl2, bl2)
    eo = _embed(xo, wo1, bo1, Wo2, bo2)[:3500]
    ee = _embed(xe, we1, be1, We2, be2)[:3500]
    x0 = jnp.concatenate(
        [eg, el, eo, ee, jnp.zeros((NPAD - N, H), f32)], axis=0)

    src = jnp.concatenate([edge_index[0], jnp.full((E_PAD - E,), N, jnp.int32)])
    dst = jnp.concatenate([edge_index[1], jnp.full((E_PAD - E,), N, jnp.int32)])
    eidx = jnp.stack([src.reshape(NW * CW, C), dst.reshape(NW * CW, C)], axis=1)

    x1 = _layer(_seg128(x0, eidx), x0, sage0_Wl, sage0_Wr, sage0_b, False)
    x2 = _layer(_seg128(x1, eidx), x1, sage1_Wl, sage1_Wr, sage1_b, False)
    x3 = _layer(_seg128(x2, eidx), x2, sage2_Wl, sage2_Wr, sage2_b, False)

    wl3 = jnp.pad(sage3_Wl, ((0, 0), (0, 7)))
    wr3 = jnp.pad(sage3_Wr, ((0, 0), (0, 7)))
    b3 = jnp.pad(sage3_b, ((0, 7),))
    y3, z3 = _head3(x3, wl3, wr3, b3)
    p3 = _seg1d(y3[:, 0], eidx)
    out = _final1d(p3.reshape(2, NPAD // H, H), z3[:, 0].reshape(NPAD // H, H))
    return out.reshape(NPAD, 1)[:N]
